# Initial kernel scaffold; baseline (speedup 1.0000x reference)
import jax
import jax.numpy as jnp
from jax.experimental import pallas as pl


def kernel(ref, query):
    raise NotImplementedError("write your pallas kernel here")


# speedup vs baseline: 12.2939x; 12.2939x over previous
import jax
import jax.numpy as jnp
from jax.experimental import pallas as pl

K = 16


def _knn_block_kernel(r_ref, q_ref, out_ref):
    r = r_ref[0]
    q = q_ref[0]
    r2 = jnp.sum(r * r, axis=0, keepdims=True)
    qr = jax.lax.dot_general(
        q, r, (((0,), (0,)), ((), ())),
        preferred_element_type=jnp.float32)
    scores = r2 - 2.0 * qr
    n = scores.shape[1]
    iota = jax.lax.broadcasted_iota(jnp.int32, scores.shape, 1)

    def body(k, s):
        m = jnp.min(s, axis=1, keepdims=True)
        idx = jnp.min(jnp.where(s == m, iota, n), axis=1)
        out_ref[0, pl.ds(k, 1), :] = idx[None, :]
        return jnp.where(iota == idx[:, None], jnp.inf, s)

    jax.lax.fori_loop(0, K, body, scores, unroll=True)


def kernel(ref, query):
    b, d, n_ref = ref.shape
    n_query = query.shape[2]
    qb = 256
    grid = (b, n_query // qb)
    return pl.pallas_call(
        _knn_block_kernel,
        grid=grid,
        in_specs=[
            pl.BlockSpec((1, d, n_ref), lambda i, j: (i, 0, 0)),
            pl.BlockSpec((1, d, qb), lambda i, j: (i, 0, j)),
        ],
        out_specs=pl.BlockSpec((1, K, qb), lambda i, j: (i, 0, j)),
        out_shape=jax.ShapeDtypeStruct((b, K, n_query), jnp.int32),
    )(ref, query)


# speedup vs baseline: 16.1768x; 1.3158x over previous
import jax
import jax.numpy as jnp
from jax.experimental import pallas as pl

K = 16


def _knn_block_kernel(r_ref, q_ref, out_ref):
    r = r_ref[0]
    q = q_ref[0]
    r2 = jnp.sum(r * r, axis=0, keepdims=True)
    qr = jax.lax.dot_general(
        q, r, (((0,), (0,)), ((), ())),
        preferred_element_type=jnp.float32)
    scores = r2 - 2.0 * qr
    n = scores.shape[1]
    iota = jax.lax.broadcasted_iota(
        jnp.int32, scores.shape, 1).astype(jnp.float32)

    def body(k, s):
        m = jnp.min(s, axis=1, keepdims=True)
        idxc = jnp.where(s == m, iota, float(n))
        idx = jnp.min(idxc, axis=1, keepdims=True)
        out_ref[0, pl.ds(k, 1), :] = idx.astype(jnp.int32).T
        return jnp.where(idxc == idx, jnp.inf, s)

    jax.lax.fori_loop(0, K, body, scores, unroll=True)


def kernel(ref, query):
    b, d, n_ref = ref.shape
    n_query = query.shape[2]
    qb = 256
    grid = (b, n_query // qb)
    return pl.pallas_call(
        _knn_block_kernel,
        grid=grid,
        in_specs=[
            pl.BlockSpec((1, d, n_ref), lambda i, j: (i, 0, 0)),
            pl.BlockSpec((1, d, qb), lambda i, j: (i, 0, j)),
        ],
        out_specs=pl.BlockSpec((1, K, qb), lambda i, j: (i, 0, j)),
        out_shape=jax.ShapeDtypeStruct((b, K, n_query), jnp.int32),
    )(ref, query)


# speedup vs baseline: 35.8599x; 2.2168x over previous
import jax
import jax.numpy as jnp
from jax.experimental import pallas as pl

K = 16


def _ce(va, ca, vb, cb):
    sw = vb < va
    return (jnp.where(sw, vb, va), jnp.where(sw, cb, ca),
            jnp.where(sw, va, vb), jnp.where(sw, ca, cb))


def _cemin(va, ca, vb, cb):
    sw = vb < va
    return jnp.where(sw, vb, va), jnp.where(sw, cb, ca)


def _merge22(a, b):
    (a1, a2), (b1, b2) = a, b
    x1, x3 = a1, b2
    x2, x4 = a2, b1
    v1, c1, v3, c3 = _ce(*x1, *x3)
    v2, c2, v4, c4 = _ce(*x2, *x4)
    v1, c1, v2, c2 = _ce(v1, c1, v2, c2)
    v3, c3, v4, c4 = _ce(v3, c3, v4, c4)
    return [(v1, c1), (v2, c2), (v3, c3), (v4, c4)]


def _merge44(a, b):
    m = [_cemin(*a[i], *b[3 - i]) for i in range(4)]
    v1, c1, v3, c3 = _ce(*m[0], *m[2])
    v2, c2, v4, c4 = _ce(*m[1], *m[3])
    v1, c1, v2, c2 = _ce(v1, c1, v2, c2)
    v3, c3, v4, c4 = _ce(v3, c3, v4, c4)
    return [(v1, c1), (v2, c2), (v3, c3), (v4, c4)]


def _full_extract(scores, out_ref):
    n = scores.shape[1]
    iota = jax.lax.broadcasted_iota(
        jnp.int32, scores.shape, 1).astype(jnp.float32)

    def body(k, s):
        m = jnp.min(s, axis=1, keepdims=True)
        idxc = jnp.where(s == m, iota, float(n))
        idx = jnp.min(idxc, axis=1, keepdims=True)
        out_ref[0, pl.ds(k, 1), :] = idx.astype(jnp.int32).T
        return jnp.where(idxc == idx, jnp.inf, s)

    jax.lax.fori_loop(0, K, body, scores, unroll=True)


def _knn_block_kernel(r_ref, q_ref, out_ref):
    r = r_ref[0]
    q = q_ref[0]
    r2 = jnp.sum(r * r, axis=0, keepdims=True)
    qr = jax.lax.dot_general(
        q, r, (((0,), (0,)), ((), ())),
        preferred_element_type=jnp.float32)
    scores = r2 - 2.0 * qr
    n = scores.shape[1]
    lane = jax.lax.broadcasted_iota(
        jnp.int32, (scores.shape[0], 128), 1).astype(jnp.float32)

    tiles = [(scores[:, t * 128:(t + 1) * 128], lane + float(t * 128))
             for t in range(n // 128)]
    s2 = []
    for i in range(0, len(tiles), 2):
        va, ca, vb, cb = _ce(*tiles[i], *tiles[i + 1])
        s2.append([(va, ca), (vb, cb)])
    s4 = [_merge22(s2[i], s2[i + 1]) for i in range(0, len(s2), 2)]
    while len(s4) > 1:
        s4 = [_merge44(s4[i], s4[i + 1]) for i in range(0, len(s4), 2)]
    cand = s4[0]
    red = jnp.concatenate([v for v, _ in cand], axis=1)
    col = jnp.concatenate([c for _, c in cand], axis=1)

    rv = red
    ms = []
    for k in range(K):
        m = jnp.min(rv, axis=1, keepdims=True)
        colc = jnp.where(rv == m, col, float(n))
        c = jnp.min(colc, axis=1, keepdims=True)
        out_ref[0, pl.ds(k, 1), :] = c.astype(jnp.int32).T
        rv = jnp.where(colc == c, jnp.inf, rv)
        ms.append(m)

    w16 = ms[-1]
    nlt = sum((m < w16).astype(jnp.float32) for m in ms)
    below = jnp.sum((scores < w16).astype(jnp.float32), axis=1,
                    keepdims=True)
    bad = jnp.any(below != nlt)
    pl.when(bad)(lambda: _full_extract(scores, out_ref))


def kernel(ref, query):
    b, d, n_ref = ref.shape
    n_query = query.shape[2]
    qb = 512
    grid = (b, n_query // qb)
    return pl.pallas_call(
        _knn_block_kernel,
        grid=grid,
        in_specs=[
            pl.BlockSpec((1, d, n_ref), lambda i, j: (i, 0, 0)),
            pl.BlockSpec((1, d, qb), lambda i, j: (i, 0, j)),
        ],
        out_specs=pl.BlockSpec((1, K, qb), lambda i, j: (i, 0, j)),
        out_shape=jax.ShapeDtypeStruct((b, K, n_query), jnp.int32),
    )(ref, query)


# speedup vs baseline: 36.9344x; 1.0300x over previous
import jax
import jax.numpy as jnp
from jax.experimental import pallas as pl

K = 16


def _ce(va, ca, vb, cb):
    sw = vb < va
    return (jnp.where(sw, vb, va), jnp.where(sw, cb, ca),
            jnp.where(sw, va, vb), jnp.where(sw, ca, cb))


def _cemin(va, ca, vb, cb):
    sw = vb < va
    return jnp.where(sw, vb, va), jnp.where(sw, cb, ca)


def _merge22(a, b):
    (a1, a2), (b1, b2) = a, b
    l1v, l1c, h1v, h1c = _ce(*a1, *b1)
    l2v, l2c, h2v, h2c = _ce(*a2, *b2)
    m1v, m1c, m2v, m2c = _ce(h1v, h1c, l2v, l2c)
    return [(l1v, l1c), (m1v, m1c), (m2v, m2c), (h2v, h2c)]


def _merge44(a, b):
    m = [_cemin(*a[i], *b[3 - i]) for i in range(4)]
    v1, c1, v3, c3 = _ce(*m[0], *m[2])
    v2, c2, v4, c4 = _ce(*m[1], *m[3])
    v1, c1, v2, c2 = _ce(v1, c1, v2, c2)
    v3, c3, v4, c4 = _ce(v3, c3, v4, c4)
    return [(v1, c1), (v2, c2), (v3, c3), (v4, c4)]


def _full_extract(scores, out_ref):
    n = scores.shape[1]
    iota = jax.lax.broadcasted_iota(
        jnp.int32, scores.shape, 1).astype(jnp.float32)

    def body(k, s):
        m = jnp.min(s, axis=1, keepdims=True)
        idxc = jnp.where(s == m, iota, float(n))
        idx = jnp.min(idxc, axis=1, keepdims=True)
        out_ref[0, pl.ds(k, 1), :] = idx.astype(jnp.int32).T
        return jnp.where(idxc == idx, jnp.inf, s)

    jax.lax.fori_loop(0, K, body, scores, unroll=True)


def _knn_block_kernel(r_ref, q_ref, out_ref):
    r = r_ref[0]
    q = q_ref[0]
    r2 = jnp.sum(r * r, axis=0, keepdims=True)
    qr = jax.lax.dot_general(
        q, r, (((0,), (0,)), ((), ())),
        preferred_element_type=jnp.float32)
    scores = r2 - 2.0 * qr
    n = scores.shape[1]
    lane = jax.lax.broadcasted_iota(
        jnp.int32, (scores.shape[0], 128), 1).astype(jnp.float32)

    tiles = [(scores[:, t * 128:(t + 1) * 128], lane + float(t * 128))
             for t in range(n // 128)]
    s2 = []
    for i in range(0, len(tiles), 2):
        va, ca, vb, cb = _ce(*tiles[i], *tiles[i + 1])
        s2.append([(va, ca), (vb, cb)])
    s4 = [_merge22(s2[i], s2[i + 1]) for i in range(0, len(s2), 2)]
    while len(s4) > 2:
        s4 = [_merge44(s4[i], s4[i + 1]) for i in range(0, len(s4), 2)]
    cand = [_cemin(*s4[0][i], *s4[1][3 - i]) for i in range(4)]
    red = jnp.concatenate([v for v, _ in cand], axis=1)
    col = jnp.concatenate([c for _, c in cand], axis=1)

    rv = red
    ms = []
    for k in range(K):
        m = jnp.min(rv, axis=1, keepdims=True)
        colc = jnp.where(rv == m, col, float(n))
        c = jnp.min(colc, axis=1, keepdims=True)
        out_ref[0, pl.ds(k, 1), :] = c.astype(jnp.int32).T
        rv = jnp.where(colc == c, jnp.inf, rv)
        ms.append(m)

    w16 = ms[-1]
    nlt = sum((m < w16).astype(jnp.float32) for m in ms)
    below = jnp.sum((scores < w16).astype(jnp.float32), axis=1,
                    keepdims=True)
    bad = jnp.any(below != nlt)
    pl.when(bad)(lambda: _full_extract(scores, out_ref))


def kernel(ref, query):
    b, d, n_ref = ref.shape
    n_query = query.shape[2]
    qb = 512
    grid = (b, n_query // qb)
    return pl.pallas_call(
        _knn_block_kernel,
        grid=grid,
        in_specs=[
            pl.BlockSpec((1, d, n_ref), lambda i, j: (i, 0, 0)),
            pl.BlockSpec((1, d, qb), lambda i, j: (i, 0, j)),
        ],
        out_specs=pl.BlockSpec((1, K, qb), lambda i, j: (i, 0, j)),
        out_shape=jax.ShapeDtypeStruct((b, K, n_query), jnp.int32),
    )(ref, query)
